# baseline (device time: 25852 ns/iter reference)
import jax
import jax.numpy as jnp
from jax import lax
from jax.experimental import pallas as pl
from jax.experimental.pallas import tpu as pltpu

M = 1024
N = 1024
H = 512
Q = 256
E = 128


def kernel(x):
    def body(x_ref, o_ref, xs, acc, rsa, rsb, xga, xgb, send_sems, recv_sems,
             copy_sems, out_sems):
        my_x = lax.axis_index("x")
        my_y = lax.axis_index("y")
        y_nbr = (my_x, 1 - my_y)
        x_nbr = (1 - my_x, my_y)

        a1 = my_y * Q
        b1 = H + my_x * Q
        a_send = (1 - my_y) * Q
        b_send = H + (1 - my_x) * Q

        def ex(src, dst, i, nbr):
            return pltpu.make_async_remote_copy(
                src_ref=src, dst_ref=dst,
                send_sem=send_sems.at[i], recv_sem=recv_sems.at[i],
                device_id=nbr, device_id_type=pl.DeviceIdType.MESH,
            )

        blocks = (a_send, b_send, a1, b1)
        copies = []
        for i, r in enumerate(blocks):
            cp = pltpu.make_async_copy(
                x_ref.at[0, 0, pl.ds(r, Q), :], xs.at[i], copy_sems.at[i])
            cp.start()
            copies.append(cp)

        barrier = pltpu.get_barrier_semaphore()
        for nbr in (y_nbr, x_nbr):
            pl.semaphore_signal(
                barrier, inc=1, device_id=nbr,
                device_id_type=pl.DeviceIdType.MESH,
            )

        copies[0].wait()
        acc[pl.ds(a_send, E), :] = xs[0, pl.ds(0, E), :].astype(jnp.bfloat16)
        a_rs1 = [ex(acc.at[pl.ds(a_send + c * E, E)], rsa.at[pl.ds(c * E, E)],
                    0 + c, y_nbr) for c in (0, 1)]
        pl.semaphore_wait(barrier, 2)
        a_rs1[0].start()
        acc[pl.ds(a_send + E, E), :] = xs[0, pl.ds(E, E), :].astype(
            jnp.bfloat16)
        a_rs1[1].start()
        copies[1].wait()
        acc[pl.ds(b_send, Q), :] = xs[1].astype(jnp.bfloat16)
        b_rs1 = [ex(acc.at[pl.ds(b_send + c * E, E)], rsb.at[pl.ds(c * E, E)],
                    6 + c, x_nbr) for c in (0, 1)]
        b_rs1[0].start()
        b_rs1[1].start()
        copies[2].wait()
        acc[pl.ds(a1, Q), :] = xs[2].astype(jnp.bfloat16)
        copies[3].wait()
        acc[pl.ds(b1, Q), :] = xs[3].astype(jnp.bfloat16)

        a_xg = [ex(acc.at[pl.ds(a1 + c * E, E)], xga.at[pl.ds(c * E, E)],
                   2 + c, x_nbr) for c in (0, 1)]
        b_xg = [ex(acc.at[pl.ds(b1 + c * E, E)], xgb.at[pl.ds(c * E, E)],
                   8 + c, y_nbr) for c in (0, 1)]
        for c in (0, 1):
            a_rs1[c].wait_recv()
            acc[pl.ds(a1 + c * E, E), :] = (
                acc[pl.ds(a1 + c * E, E), :] + rsa[pl.ds(c * E, E), :])
            a_xg[c].start()
            b_rs1[c].wait_recv()
            acc[pl.ds(b1 + c * E, E), :] = (
                acc[pl.ds(b1 + c * E, E), :] + rsb[pl.ds(c * E, E), :])
            b_xg[c].start()

        a_ag = [ex(acc.at[pl.ds(a1 + c * E, E)], o_ref.at[pl.ds(a1 + c * E, E)],
                   4 + c, y_nbr) for c in (0, 1)]
        b_ag = [ex(acc.at[pl.ds(b1 + c * E, E)], o_ref.at[pl.ds(b1 + c * E, E)],
                   10 + c, x_nbr) for c in (0, 1)]
        out_copies = []
        for c in (0, 1):
            a_xg[c].wait_recv()
            acc[pl.ds(a1 + c * E, E), :] = (
                acc[pl.ds(a1 + c * E, E), :] + xga[pl.ds(c * E, E), :])
            a_ag[c].start()
            cp = pltpu.make_async_copy(
                acc.at[pl.ds(a1 + c * E, E)], o_ref.at[pl.ds(a1 + c * E, E)],
                out_sems.at[2 * c])
            cp.start()
            out_copies.append(cp)
            b_xg[c].wait_recv()
            acc[pl.ds(b1 + c * E, E), :] = (
                acc[pl.ds(b1 + c * E, E), :] + xgb[pl.ds(c * E, E), :])
            b_ag[c].start()
            cp = pltpu.make_async_copy(
                acc.at[pl.ds(b1 + c * E, E)], o_ref.at[pl.ds(b1 + c * E, E)],
                out_sems.at[2 * c + 1])
            cp.start()
            out_copies.append(cp)

        for c in (0, 1):
            a_ag[c].wait_recv()
            b_ag[c].wait_recv()
        for cp in out_copies:
            cp.wait()
        for d in a_rs1 + b_rs1 + a_xg + b_xg + a_ag + b_ag:
            d.wait_send()

    return pl.pallas_call(
        body,
        out_shape=jax.ShapeDtypeStruct((M, N), jnp.bfloat16),
        in_specs=[pl.BlockSpec(memory_space=pltpu.MemorySpace.HBM)],
        out_specs=pl.BlockSpec(memory_space=pltpu.MemorySpace.HBM),
        scratch_shapes=[
            pltpu.VMEM((4, Q, N), jnp.float32),
            pltpu.VMEM((M, N), jnp.bfloat16),
            pltpu.VMEM((Q, N), jnp.bfloat16),
            pltpu.VMEM((Q, N), jnp.bfloat16),
            pltpu.VMEM((Q, N), jnp.bfloat16),
            pltpu.VMEM((Q, N), jnp.bfloat16),
            pltpu.SemaphoreType.DMA((12,)),
            pltpu.SemaphoreType.DMA((12,)),
            pltpu.SemaphoreType.DMA((4,)),
            pltpu.SemaphoreType.DMA((4,)),
        ],
        compiler_params=pltpu.CompilerParams(collective_id=0),
    )(x)


# device time: 25455 ns/iter; 1.0156x vs baseline; 1.0156x over previous
import jax
import jax.numpy as jnp
from jax import lax
from jax.experimental import pallas as pl
from jax.experimental.pallas import tpu as pltpu

M = 1024
N = 1024
H = 512
Q = 256
E = 128


def kernel(x):
    def body(x_ref, o_ref, rsa, rsb, xga, xgb, send_sems, recv_sems):
        my_x = lax.axis_index("x")
        my_y = lax.axis_index("y")
        y_nbr = (my_x, 1 - my_y)
        x_nbr = (1 - my_x, my_y)

        a1 = my_y * Q
        b1 = H + my_x * Q
        a_send = (1 - my_y) * Q
        b_send = H + (1 - my_x) * Q

        def ex(src, dst, i, nbr):
            return pltpu.make_async_remote_copy(
                src_ref=src, dst_ref=dst,
                send_sem=send_sems.at[i], recv_sem=recv_sems.at[i],
                device_id=nbr, device_id_type=pl.DeviceIdType.MESH,
            )

        barrier = pltpu.get_barrier_semaphore()
        for nbr in (y_nbr, x_nbr):
            pl.semaphore_signal(
                barrier, inc=1, device_id=nbr,
                device_id_type=pl.DeviceIdType.MESH,
            )

        o_ref[pl.ds(a_send, E), :] = x_ref[0, 0, pl.ds(a_send, E), :].astype(
            jnp.bfloat16)
        a_rs1 = [ex(o_ref.at[pl.ds(a_send + c * E, E)], rsa.at[pl.ds(c * E, E)],
                    0 + c, y_nbr) for c in (0, 1)]
        pl.semaphore_wait(barrier, 2)
        a_rs1[0].start()
        o_ref[pl.ds(a_send + E, E), :] = x_ref[
            0, 0, pl.ds(a_send + E, E), :].astype(jnp.bfloat16)
        a_rs1[1].start()
        o_ref[pl.ds(b_send, Q), :] = x_ref[0, 0, pl.ds(b_send, Q), :].astype(
            jnp.bfloat16)
        b_rs1 = [ex(o_ref.at[pl.ds(b_send + c * E, E)], rsb.at[pl.ds(c * E, E)],
                    6 + c, x_nbr) for c in (0, 1)]
        b_rs1[0].start()
        b_rs1[1].start()
        o_ref[pl.ds(a1, Q), :] = x_ref[0, 0, pl.ds(a1, Q), :].astype(
            jnp.bfloat16)
        o_ref[pl.ds(b1, Q), :] = x_ref[0, 0, pl.ds(b1, Q), :].astype(
            jnp.bfloat16)

        a_xg = [ex(o_ref.at[pl.ds(a1 + c * E, E)], xga.at[pl.ds(c * E, E)],
                   2 + c, x_nbr) for c in (0, 1)]
        b_xg = [ex(o_ref.at[pl.ds(b1 + c * E, E)], xgb.at[pl.ds(c * E, E)],
                   8 + c, y_nbr) for c in (0, 1)]
        for c in (0, 1):
            a_rs1[c].wait_recv()
            o_ref[pl.ds(a1 + c * E, E), :] = (
                o_ref[pl.ds(a1 + c * E, E), :] + rsa[pl.ds(c * E, E), :])
            a_xg[c].start()
            b_rs1[c].wait_recv()
            o_ref[pl.ds(b1 + c * E, E), :] = (
                o_ref[pl.ds(b1 + c * E, E), :] + rsb[pl.ds(c * E, E), :])
            b_xg[c].start()

        a_ag = [ex(o_ref.at[pl.ds(a1 + c * E, E)], o_ref.at[pl.ds(a1 + c * E, E)],
                   4 + c, y_nbr) for c in (0, 1)]
        b_ag = [ex(o_ref.at[pl.ds(b1 + c * E, E)], o_ref.at[pl.ds(b1 + c * E, E)],
                   10 + c, x_nbr) for c in (0, 1)]
        for c in (0, 1):
            a_xg[c].wait_recv()
            o_ref[pl.ds(a1 + c * E, E), :] = (
                o_ref[pl.ds(a1 + c * E, E), :] + xga[pl.ds(c * E, E), :])
            a_ag[c].start()
            b_xg[c].wait_recv()
            o_ref[pl.ds(b1 + c * E, E), :] = (
                o_ref[pl.ds(b1 + c * E, E), :] + xgb[pl.ds(c * E, E), :])
            b_ag[c].start()

        for c in (0, 1):
            a_ag[c].wait_recv()
            b_ag[c].wait_recv()

        for d in a_rs1 + b_rs1 + a_xg + b_xg + a_ag + b_ag:
            d.wait_send()

    return pl.pallas_call(
        body,
        out_shape=jax.ShapeDtypeStruct((M, N), jnp.bfloat16),
        in_specs=[pl.BlockSpec(memory_space=pltpu.VMEM)],
        out_specs=pl.BlockSpec(memory_space=pltpu.VMEM),
        scratch_shapes=[
            pltpu.VMEM((Q, N), jnp.bfloat16),
            pltpu.VMEM((Q, N), jnp.bfloat16),
            pltpu.VMEM((Q, N), jnp.bfloat16),
            pltpu.VMEM((Q, N), jnp.bfloat16),
            pltpu.SemaphoreType.DMA((12,)),
            pltpu.SemaphoreType.DMA((12,)),
        ],
        compiler_params=pltpu.CompilerParams(collective_id=0),
    )(x)
